# Initial kernel scaffold; baseline (speedup 1.0000x reference)
import jax
import jax.numpy as jnp
from jax.experimental import pallas as pl


def kernel(x, W1, g1, b1, W2, g2, b2, W3, g3, b3, W4, g4, b4, W5, g5, b5, Wp1, bp1, gp, bp, Wp2, bp2):
    raise NotImplementedError("write your pallas kernel here")


# speedup vs baseline: 7.6442x; 7.6442x over previous
import functools

import jax
import jax.numpy as jnp
from jax import lax
from jax.experimental import pallas as pl
from jax.experimental.pallas import tpu as pltpu
from jax.experimental.pallas import tpu_sc as plsc

K = 20
EPS = 1e-5
B = 4
N = 2048
P = B * N

TN = 256
TS = 512

_NEG = float('-inf')


def _dot_nt(a, b):
    return lax.dot_general(a, b, (((1,), (1,)), ((), ())),
                           preferred_element_type=jnp.float32)


def _knn_body(xt_tile_ref, xt_full_ref, xtt_ref, idx_ref):
    b = pl.program_id(0)
    xt_t = xt_tile_ref[0]
    xt_f = xt_full_ref[0]
    xtt = xtt_ref[0]
    g = _dot_nt(xt_t.astype(jnp.bfloat16), xt_f.astype(jnp.bfloat16))
    sq_t = jnp.sum(xt_t * xt_t, axis=1, keepdims=True)
    sq_f = jnp.sum(xtt * xtt, axis=0, keepdims=True)
    pd = 2.0 * g - sq_t - sq_f
    iota = lax.broadcasted_iota(jnp.int32, (TN, N), 1)
    cols = []
    for _ in range(K):
        rm = jnp.max(pd, axis=1, keepdims=True)
        cand = jnp.where(pd == rm, iota, N)
        sel = jnp.min(cand, axis=1, keepdims=True)
        cols.append(sel)
        pd = jnp.where(iota == sel, _NEG, pd)
    idx = jnp.concatenate(cols, axis=1) + b * N
    idx_ref[0] = idx


def _knn(xt):
    d = xt.shape[2]
    xtt = jnp.swapaxes(xt, 1, 2)
    return pl.pallas_call(
        _knn_body,
        grid=(B, N // TN),
        in_specs=[
            pl.BlockSpec((1, TN, d), lambda b, i: (b, i, 0)),
            pl.BlockSpec((1, N, d), lambda b, i: (b, 0, 0)),
            pl.BlockSpec((1, d, N), lambda b, i: (b, 0, 0)),
        ],
        out_specs=pl.BlockSpec((1, TN, K), lambda b, i: (b, i, 0)),
        out_shape=jax.ShapeDtypeStruct((B, N, K), jnp.int32),
    )(xt, xt, xtt)


NW = 32
PTS_W = P // NW
CH = 16
NCH = PTS_W // CH
ROWS = CH * K
GSZ = 80
NG = ROWS // GSZ
DS = 128


def _gather(x_flat, idx_flat):
    mesh = plsc.VectorSubcoreMesh(core_axis_name="c", subcore_axis_name="s")

    def body(x_hbm, idx_hbm, rows_hbm, idx_v, rows_a, rows_b, sem):
        wid = lax.axis_index("s") * 2 + lax.axis_index("c")

        def fire(c, rows_v):
            r0 = (wid * PTS_W + c * CH) * K
            pltpu.sync_copy(idx_hbm.at[pl.ds(r0, ROWS)],
                            idx_v.at[pl.ds((c % 2) * ROWS, ROWS)])
            cps = []
            for gch in range(NG):
                cps.append(pltpu.async_copy(
                    x_hbm.at[idx_v.at[pl.ds((c % 2) * ROWS + gch * GSZ, GSZ)]],
                    rows_v.at[pl.ds(gch * GSZ, GSZ)], sem))
            return cps

        def drain(c, rows_v, cps):
            for cp in cps:
                cp.wait()
            r0 = (wid * PTS_W + c * CH) * K
            pltpu.sync_copy(rows_v, rows_hbm.at[pl.ds(r0, ROWS)])

        cps = fire(0, rows_a)
        for c in range(NCH):
            nxt = None
            if c + 1 < NCH:
                nxt = fire(c + 1, rows_b if (c % 2 == 0) else rows_a)
            drain(c, rows_a if (c % 2 == 0) else rows_b, cps)
            cps = nxt

    kern = functools.partial(
        pl.kernel,
        out_type=jax.ShapeDtypeStruct((P * K, DS), jnp.float32),
        mesh=mesh,
        scratch_types=[
            pltpu.VMEM((2 * ROWS,), jnp.int32),
            pltpu.VMEM((ROWS, DS), jnp.float32),
            pltpu.VMEM((ROWS, DS), jnp.float32),
            pltpu.SemaphoreType.DMA,
        ],
    )(body)
    return kern(x_flat, idx_flat)


TNE = 128
NPROG = P // TNE


def _edge_body(rows_ref, ctr_ref, w_ref, gmax_ref, s_ref, q_ref):
    ctr = ctr_ref[...]
    rows3 = rows_ref[...].reshape(TNE, K, DS)
    ctr3 = ctr[:, None, :]
    fn = (rows3 - ctr3).astype(jnp.bfloat16)
    cr = jnp.broadcast_to(ctr3, rows3.shape).astype(jnp.bfloat16)
    f = jnp.concatenate([fn, cr], axis=2).reshape(TNE * K, 2 * DS)
    h = _dot_nt(f, w_ref[...].astype(jnp.bfloat16))
    h3 = h.reshape(TNE, K, h.shape[1])
    hk = h3[:, 0, :]
    gmax = hk
    s_acc = hk
    q_acc = hk * hk
    for k in range(1, K):
        hk = h3[:, k, :]
        gmax = jnp.maximum(gmax, hk)
        s_acc = s_acc + hk
        q_acc = q_acc + hk * hk
    gmax_ref[...] = gmax
    s_ref[0] = jnp.sum(s_acc, axis=0, keepdims=True)
    q_ref[0] = jnp.sum(q_acc, axis=0, keepdims=True)


def _edge(rows, x_flat, w, o):
    return pl.pallas_call(
        _edge_body,
        grid=(NPROG,),
        in_specs=[
            pl.BlockSpec((TNE * K, DS), lambda i: (i, 0)),
            pl.BlockSpec((TNE, DS), lambda i: (i, 0)),
            pl.BlockSpec((o, 2 * DS), lambda i: (0, 0)),
        ],
        out_specs=[
            pl.BlockSpec((TNE, o), lambda i: (i, 0)),
            pl.BlockSpec((1, 1, o), lambda i: (i, 0, 0)),
            pl.BlockSpec((1, 1, o), lambda i: (i, 0, 0)),
        ],
        out_shape=[
            jax.ShapeDtypeStruct((P, o), jnp.float32),
            jax.ShapeDtypeStruct((NPROG, 1, o), jnp.float32),
            jax.ShapeDtypeStruct((NPROG, 1, o), jnp.float32),
        ],
    )(rows, x_flat, w)


def _finish_body(gmax_ref, s_ref, q_ref, g_ref, bb_ref, out_ref):
    cnt = float(P * K)
    m = jnp.sum(s_ref[...], axis=0, keepdims=True) / cnt
    var = jnp.sum(q_ref[...], axis=0, keepdims=True) / cnt - m * m
    out = (gmax_ref[...] - m) / jnp.sqrt(var + EPS) * g_ref[...] + bb_ref[...]
    out_ref[...] = jnp.where(out >= 0, out, 0.2 * out)


def _finish(gmax, s, q, g, bb, o):
    vspec = pl.BlockSpec((1, o), lambda i: (0, 0))
    return pl.pallas_call(
        _finish_body,
        grid=(P // TS,),
        in_specs=[
            pl.BlockSpec((TS, o), lambda i: (i, 0)),
            pl.BlockSpec((NPROG, o), lambda i: (0, 0)),
            pl.BlockSpec((NPROG, o), lambda i: (0, 0)),
            vspec, vspec,
        ],
        out_specs=pl.BlockSpec((TS, o), lambda i: (i, 0)),
        out_shape=jax.ShapeDtypeStruct((P, o), jnp.float32),
    )(gmax, s, q, g, bb)


def _pad_to(a, shape):
    return jnp.pad(a, [(0, t - s) for s, t in zip(a.shape, shape)])


def _edge_layer(xt, w, g, bb, o_pad):
    o, two_d = w.shape
    d = two_d // 2
    wa = _pad_to(w[:, :d], (o_pad, DS))
    wb = _pad_to(w[:, d:], (o_pad, DS))
    wcat = jnp.concatenate([wa, wb], axis=1)
    g = _pad_to(g, (o_pad,)).reshape(1, o_pad)
    bb = _pad_to(bb, (o_pad,)).reshape(1, o_pad)
    o = o_pad
    idx = _knn(xt)
    x_flat = xt.reshape(P, DS)
    rows = _gather(x_flat, idx.reshape(P * K))
    gmax, s, q = _edge(rows, x_flat, wcat, o)
    out = _finish(gmax, s.reshape(NPROG, o), q.reshape(NPROG, o), g, bb, o)
    return out.reshape(B, N, o)


def _f1_body(x1_ref, x2_ref, x3_ref, x4_ref, w51_ref, w52_ref, w53_ref,
             w54_ref, h_ref, s_ref, q_ref):
    i = pl.program_id(0)
    h = (_dot_nt(x1_ref[...], w51_ref[...]) +
         _dot_nt(x2_ref[...], w52_ref[...]) +
         _dot_nt(x3_ref[...], w53_ref[...]) +
         _dot_nt(x4_ref[...], w54_ref[...]))
    h_ref[...] = h

    @pl.when(i == 0)
    def _():
        s_ref[...] = jnp.sum(h, axis=0, keepdims=True)
        q_ref[...] = jnp.sum(h * h, axis=0, keepdims=True)

    @pl.when(i != 0)
    def _():
        s_ref[...] += jnp.sum(h, axis=0, keepdims=True)
        q_ref[...] += jnp.sum(h * h, axis=0, keepdims=True)


def _f2_body(h_ref, s_ref, q_ref, g_ref, bb_ref, out_ref):
    i = pl.program_id(0)
    cnt = float(P)
    m = s_ref[...] / cnt
    var = q_ref[...] / cnt - m * m
    scale = g_ref[...] / jnp.sqrt(var + EPS)
    h = (h_ref[...] - m) * scale + bb_ref[...]
    h = jnp.where(h >= 0, h, 0.2 * h)
    tmax = jnp.max(h, axis=0, keepdims=True)

    @pl.when(i % (N // TS) == 0)
    def _():
        out_ref[0] = tmax

    @pl.when(i % (N // TS) != 0)
    def _():
        out_ref[0] = jnp.maximum(out_ref[0], tmax)


def _f3_body(hg_ref, wp1_ref, bp1_ref, gp_ref, bp_ref, wp2_ref, bp2_ref,
             z_ref):
    p = _dot_nt(hg_ref[...], wp1_ref[...]) + bp1_ref[...]
    m = jnp.mean(p, axis=0, keepdims=True)
    v = jnp.mean((p - m) * (p - m), axis=0, keepdims=True)
    p = (p - m) / jnp.sqrt(v + EPS) * gp_ref[...] + bp_ref[...]
    p = jnp.maximum(p, 0.0)
    z_ref[...] = _dot_nt(p, wp2_ref[...]) + bp2_ref[...]


def _final(x1, x2, x3, x4, w5, g5, b5, wp1, bp1, gp, bp, wp2, bp2):
    xf = [x.reshape(P, -1) for x in (x1, x2, x3, x4)]
    w5s = [
        _pad_to(w5[:, 0:64], (512, xf[0].shape[1])),
        _pad_to(w5[:, 64:128], (512, xf[1].shape[1])),
        w5[:, 128:256],
        w5[:, 256:512],
    ]
    vec = pl.BlockSpec((1, 512), lambda i: (0, 0))
    h, s, q = pl.pallas_call(
        _f1_body,
        grid=(P // TS,),
        in_specs=[
            pl.BlockSpec((TS, xf[0].shape[1]), lambda i: (i, 0)),
            pl.BlockSpec((TS, xf[1].shape[1]), lambda i: (i, 0)),
            pl.BlockSpec((TS, 128), lambda i: (i, 0)),
            pl.BlockSpec((TS, 256), lambda i: (i, 0)),
            pl.BlockSpec((512, xf[0].shape[1]), lambda i: (0, 0)),
            pl.BlockSpec((512, xf[1].shape[1]), lambda i: (0, 0)),
            pl.BlockSpec((512, 128), lambda i: (0, 0)),
            pl.BlockSpec((512, 256), lambda i: (0, 0)),
        ],
        out_specs=[pl.BlockSpec((TS, 512), lambda i: (i, 0)), vec, vec],
        out_shape=[
            jax.ShapeDtypeStruct((P, 512), jnp.float32),
            jax.ShapeDtypeStruct((1, 512), jnp.float32),
            jax.ShapeDtypeStruct((1, 512), jnp.float32),
        ],
    )(*xf, *w5s)
    hg = pl.pallas_call(
        _f2_body,
        grid=(P // TS,),
        in_specs=[
            pl.BlockSpec((TS, 512), lambda i: (i, 0)),
            vec, vec, vec, vec,
        ],
        out_specs=pl.BlockSpec((1, 1, 512), lambda i: (i // (N // TS), 0, 0)),
        out_shape=jax.ShapeDtypeStruct((B, 1, 512), jnp.float32),
    )(h, s, q, g5.reshape(1, 512), b5.reshape(1, 512))
    hg = hg.reshape(B, 512)
    return pl.pallas_call(
        _f3_body,
        in_specs=[
            pl.BlockSpec((B, 512), lambda: (0, 0)),
            pl.BlockSpec((256, 512), lambda: (0, 0)),
            pl.BlockSpec((1, 256), lambda: (0, 0)),
            pl.BlockSpec((1, 256), lambda: (0, 0)),
            pl.BlockSpec((1, 256), lambda: (0, 0)),
            pl.BlockSpec((128, 256), lambda: (0, 0)),
            pl.BlockSpec((1, 128), lambda: (0, 0)),
        ],
        out_specs=pl.BlockSpec((B, 128), lambda: (0, 0)),
        out_shape=jax.ShapeDtypeStruct((B, 128), jnp.float32),
    )(hg, wp1, bp1.reshape(1, 256), gp.reshape(1, 256), bp.reshape(1, 256),
      wp2, bp2.reshape(1, 128))


def kernel(x, W1, g1, b1, W2, g2, b2, W3, g3, b3, W4, g4, b4, W5, g5, b5,
           Wp1, bp1, gp, bp, Wp2, bp2):
    x = _pad_to(x, (B, N, DS))
    x1 = _edge_layer(x, W1, g1, b1, 128)
    x2 = _edge_layer(x1, W2, g2, b2, 128)
    x3 = _edge_layer(x2, W3, g3, b3, 128)
    x4 = _edge_layer(x3, W4, g4, b4, 256)
    return _final(x1, x2, x3, x4, W5, g5, b5, Wp1, bp1, gp, bp, Wp2, bp2)
